# baseline (speedup 1.0000x reference)
import functools

import jax
import jax.numpy as jnp
from jax import lax
from jax.experimental import pallas as pl
from jax.experimental.pallas import tpu as pltpu
from jax.experimental.pallas import tpu_sc as plsc


def kernel(inputs, pos_table):
    N, T = inputs.shape
    H = pos_table.shape[1]
    D = T * H
    scale = float(H) ** 0.5

    NC, NS = 2, 16
    NW = NC * NS
    K = 4
    CH = N // K
    rows_per_w = CH // NW

    mesh = plsc.VectorSubcoreMesh(core_axis_name="c", subcore_axis_name="s")

    def make_chunk_kernel(chunk_idx):
        @functools.partial(
            pl.kernel,
            mesh=mesh,
            out_type=jax.ShapeDtypeStruct((CH, D), jnp.float32),
            scratch_types=[
                pltpu.VMEM((D,), jnp.float32),
                pltpu.SemaphoreType.DMA,
            ],
        )
        def sc_chunk(tab_hbm, out_hbm, buf, sem):
            wid = lax.axis_index("s") * NC + lax.axis_index("c")
            pltpu.sync_copy(tab_hbm, buf)

            @pl.loop(0, D, step=16)
            def _(i):
                buf.at[pl.ds(i, 16)][...] = buf.at[pl.ds(i, 16)][...] * scale

            base = ((wid + chunk_idx) % NW) * rows_per_w

            @pl.loop(0, rows_per_w)
            def _(r):
                pltpu.async_copy(buf, out_hbm.at[base + r], sem)

            @pl.loop(0, rows_per_w)
            def _(r):
                pltpu.make_async_copy(buf, out_hbm.at[base + r], sem).wait()

        return sc_chunk

    tab_flat = pos_table.reshape(D)
    out = jnp.zeros((N, T, H), dtype=jnp.float32)
    for i in range(K):
        part = make_chunk_kernel(i)(tab_flat).reshape(CH, T, H)
        out = jax.lax.dynamic_update_slice(out, part, (i * CH, 0, 0))
    return out


# speedup vs baseline: 1.2513x; 1.2513x over previous
import functools

import jax
import jax.numpy as jnp
from jax import lax
from jax.experimental import pallas as pl
from jax.experimental.pallas import tpu as pltpu
from jax.experimental.pallas import tpu_sc as plsc


def kernel(inputs, pos_table):
    N, T = inputs.shape
    H = pos_table.shape[1]
    D = T * H
    scale = float(H) ** 0.5

    NC, NS = 2, 16
    NW = NC * NS
    N_TC = N // 2
    N_SC = N - N_TC
    rows_per_w = N_SC // NW

    BN = 128

    def tc_body(tab_ref, out_ref):
        out_ref[...] = jnp.broadcast_to(tab_ref[...] * scale, out_ref.shape)

    tc_half = pl.pallas_call(
        tc_body,
        grid=(N_TC // BN,),
        in_specs=[pl.BlockSpec((1, D), lambda i: (0, 0))],
        out_specs=pl.BlockSpec((BN, D), lambda i: (i, 0)),
        out_shape=jax.ShapeDtypeStruct((N_TC, D), jnp.float32),
    )(pos_table.reshape(1, D))

    mesh = plsc.VectorSubcoreMesh(core_axis_name="c", subcore_axis_name="s")

    @functools.partial(
        pl.kernel,
        mesh=mesh,
        out_type=jax.ShapeDtypeStruct((N_SC, D), jnp.float32),
        scratch_types=[
            pltpu.VMEM((D,), jnp.float32),
            pltpu.SemaphoreType.DMA,
        ],
    )
    def sc_replicate(tab_hbm, out_hbm, buf, sem):
        wid = lax.axis_index("s") * NC + lax.axis_index("c")
        pltpu.sync_copy(tab_hbm, buf)

        @pl.loop(0, D, step=16)
        def _(i):
            buf.at[pl.ds(i, 16)][...] = buf.at[pl.ds(i, 16)][...] * scale

        base = wid * rows_per_w

        @pl.loop(0, rows_per_w)
        def _(r):
            pltpu.async_copy(buf, out_hbm.at[base + r], sem)

        @pl.loop(0, rows_per_w)
        def _(r):
            pltpu.make_async_copy(buf, out_hbm.at[base + r], sem).wait()

    sc_half = sc_replicate(pos_table.reshape(D))

    flat = jnp.concatenate([tc_half, sc_half], axis=0)
    return flat.reshape(N, T, H)


# speedup vs baseline: 1.2757x; 1.0195x over previous
import functools

import jax
import jax.numpy as jnp
from jax import lax
from jax.experimental import pallas as pl
from jax.experimental.pallas import tpu as pltpu
from jax.experimental.pallas import tpu_sc as plsc


def kernel(inputs, pos_table):
    N, T = inputs.shape
    H = pos_table.shape[1]
    D = T * H
    scale = float(H) ** 0.5

    NC, NS = 2, 16
    NW = NC * NS
    N_TC = 3 * N // 4
    N_SC = N - N_TC
    rows_per_w = N_SC // NW

    BN = 128

    def tc_body(tab_ref, out_ref):
        out_ref[...] = jnp.broadcast_to(tab_ref[...] * scale, out_ref.shape)

    tc_half = pl.pallas_call(
        tc_body,
        grid=(N_TC // BN,),
        in_specs=[pl.BlockSpec((1, D), lambda i: (0, 0))],
        out_specs=pl.BlockSpec((BN, D), lambda i: (i, 0)),
        out_shape=jax.ShapeDtypeStruct((N_TC, D), jnp.float32),
    )(pos_table.reshape(1, D))

    mesh = plsc.VectorSubcoreMesh(core_axis_name="c", subcore_axis_name="s")

    @functools.partial(
        pl.kernel,
        mesh=mesh,
        out_type=jax.ShapeDtypeStruct((N_SC, D), jnp.float32),
        scratch_types=[
            pltpu.VMEM((D,), jnp.float32),
            pltpu.SemaphoreType.DMA,
        ],
    )
    def sc_replicate(tab_hbm, out_hbm, buf, sem):
        wid = lax.axis_index("s") * NC + lax.axis_index("c")
        pltpu.sync_copy(tab_hbm, buf)

        @pl.loop(0, D, step=16)
        def _(i):
            buf.at[pl.ds(i, 16)][...] = buf.at[pl.ds(i, 16)][...] * scale

        base = wid * rows_per_w

        @pl.loop(0, rows_per_w)
        def _(r):
            pltpu.async_copy(buf, out_hbm.at[base + r], sem)

        @pl.loop(0, rows_per_w)
        def _(r):
            pltpu.make_async_copy(buf, out_hbm.at[base + r], sem).wait()

    sc_half = sc_replicate(pos_table.reshape(D))

    flat = jnp.concatenate([tc_half, sc_half], axis=0)
    return flat.reshape(N, T, H)
